# baseline (speedup 1.0000x reference)
import functools

import jax
import jax.numpy as jnp
from jax import lax
from jax.experimental import pallas as pl
from jax.experimental.pallas import tpu as pltpu
from jax.experimental.pallas import tpu_sc as plsc

B, S = 4096, 200
TABLE_SIZE = 1000000
MULT = 2654435761
M1 = MULT & 0xFFFFFFFF
M2 = (MULT * MULT) & 0xFFFFFFFF
M3 = (MULT * MULT * MULT) & 0xFFFFFFFF

NTOT = B * S
NW = 32
CHUNK = NTOT // NW
ROWS = NTOT // 128
HASH_BLK = 800


def _hash_body(x_ref, out):
    xb = x_ref[...].astype(jnp.uint32)
    zrow = jnp.zeros((1, 128), jnp.uint32)
    xprev = jnp.concatenate([zrow, xb[:-1, :]], axis=0)

    def shift(k):
        return jnp.concatenate([xprev[:, 128 - k:], xb[:, :128 - k]], axis=1)

    r = jax.lax.broadcasted_iota(jnp.uint32, (ROWS, 128), 0)
    l = jax.lax.broadcasted_iota(jnp.uint32, (ROWS, 128), 1)
    pm = (r * jnp.uint32(128) + l) % jnp.uint32(S)
    zero = jnp.uint32(0)
    a0 = xb
    a1 = jnp.where(pm >= jnp.uint32(1), shift(1), zero)
    a2 = jnp.where(pm >= jnp.uint32(2), shift(2), zero)
    a3 = jnp.where(pm >= jnp.uint32(3), shift(3), zero)
    ts = jnp.uint32(TABLE_SIZE)
    h2 = a1 * jnp.uint32(M1) + a0
    h3 = a2 * jnp.uint32(M2) + h2
    h4 = a3 * jnp.uint32(M3) + h3
    out[0] = (h2 % ts).astype(jnp.int32)
    out[1] = (h3 % ts).astype(jnp.int32)
    out[2] = (h4 % ts).astype(jnp.int32)


def _compute_indices(xf):
    return pl.pallas_call(
        _hash_body,
        out_shape=jax.ShapeDtypeStruct((3, ROWS, 128), jnp.int32),
    )(xf)


SEG = 62496
TAIL = TABLE_SIZE - 15 * SEG
NSTG = 10
NBNC = 3
STG = TAIL // NSTG
NB = 4
GB = CHUNK // NB


def _gather_body(idx_h, tb2_h, tb3_h, tb4_h, out_h,
                 idx_v0, idx_v1, out_v0, out_v1, bnc_v0, bnc_v1, bnc_v2, tb_s,
                 sem_h, sem_s, sem_i, sem_g, sem_o):
    c = lax.axis_index("c")
    s = lax.axis_index("s")
    wid = s * 2 + c
    base = wid * CHUNK
    idx_bufs = (idx_v0, idx_v1)
    out_bufs = (out_v0, out_v1)
    bncs = (bnc_v0, bnc_v1, bnc_v2)
    off = jnp.minimum(s * SEG, TABLE_SIZE - TAIL)
    tabs = (tb2_h, tb3_h, tb4_h)
    idx_descs = {}
    last_out = {0: None, 1: None}

    def idx_start(t, b):
        o = t * NTOT + base + b * GB
        idx_descs[(t, b)] = pltpu.async_copy(
            idx_h.at[pl.ds(o, GB)], idx_bufs[b % 2], sem_i
        )

    idx_start(0, 0)
    for t in range(3):
        tb_h = tabs[t]
        if t > 0:
            plsc.subcore_barrier()

        def stg_h(r):
            return pltpu.async_copy(
                tb_h.at[pl.ds(off + r * STG, STG)], bncs[r % NBNC], sem_h
            )

        hd = [None] * NSTG
        sd = [None] * NSTG
        for r in range(min(NBNC, NSTG)):
            hd[r] = stg_h(r)
        for r in range(NSTG):
            hd[r].wait()
            sd[r] = pltpu.async_copy(
                bncs[r % NBNC], tb_s.at[pl.ds(off + r * STG, STG)], sem_s
            )
            if r + NBNC < NSTG:
                sd[r].wait()
                hd[r + NBNC] = stg_h(r + NBNC)
        for r in range(max(0, NSTG - NBNC), NSTG):
            sd[r].wait()
        plsc.subcore_barrier()

        def out_start(b):
            o = t * NTOT + base + b * GB
            last_out[b % 2] = pltpu.async_copy(
                out_bufs[b % 2], out_h.at[pl.ds(o, GB)], sem_o
            )

        gd = {}
        for b in range(NB):
            idx_descs[(t, b)].wait()
            p = b % 2
            if last_out[p] is not None:
                last_out[p].wait()
            gd[b] = pltpu.async_copy(tb_s.at[idx_bufs[p]], out_bufs[p], sem_g)
            if b > 0:
                gd[b - 1].wait()
                out_start(b - 1)
            if b + 1 < NB:
                idx_start(t, b + 1)
            elif t < 2:
                idx_start(t + 1, 0)
        gd[NB - 1].wait()
        out_start(NB - 1)
    last_out[0].wait()
    last_out[1].wait()


@functools.cache
def _gather():
    return functools.partial(
        pl.kernel,
        out_type=jax.ShapeDtypeStruct((3 * NTOT,), jnp.float32),
        mesh=plsc.VectorSubcoreMesh(core_axis_name="c", subcore_axis_name="s"),
        scratch_types=[
            pltpu.VMEM((GB,), jnp.int32),
            pltpu.VMEM((GB,), jnp.int32),
            pltpu.VMEM((GB,), jnp.float32),
            pltpu.VMEM((GB,), jnp.float32),
            pltpu.VMEM((STG,), jnp.float32),
            pltpu.VMEM((STG,), jnp.float32),
            pltpu.VMEM((STG,), jnp.float32),
            pltpu.VMEM_SHARED((TABLE_SIZE,), jnp.float32),
            pltpu.SemaphoreType.DMA,
            pltpu.SemaphoreType.DMA,
            pltpu.SemaphoreType.DMA,
            pltpu.SemaphoreType.DMA,
            pltpu.SemaphoreType.DMA,
        ],
    )(_gather_body)


def kernel(x, table_2, table_3, table_4):
    xf = x.reshape(ROWS, 128)
    idx = _compute_indices(xf).reshape(3 * NTOT)
    out = _gather()(idx, table_2, table_3, table_4)
    return out.reshape(3, B, S)


# speedup vs baseline: 1.0043x; 1.0043x over previous
import functools

import jax
import jax.numpy as jnp
from jax import lax
from jax.experimental import pallas as pl
from jax.experimental.pallas import tpu as pltpu
from jax.experimental.pallas import tpu_sc as plsc

B, S = 4096, 200
TABLE_SIZE = 1000000
MULT = 2654435761
M1 = MULT & 0xFFFFFFFF
M2 = (MULT * MULT) & 0xFFFFFFFF
M3 = (MULT * MULT * MULT) & 0xFFFFFFFF

NTOT = B * S
NW = 32
CHUNK = NTOT // NW
ROWS = NTOT // 128
HASH_BLK = 800


def _hash_body(x_ref, out):
    xb = x_ref[...].astype(jnp.uint32)
    zrow = jnp.zeros((1, 128), jnp.uint32)
    xprev = jnp.concatenate([zrow, xb[:-1, :]], axis=0)

    def shift(k):
        return jnp.concatenate([xprev[:, 128 - k:], xb[:, :128 - k]], axis=1)

    r = jax.lax.broadcasted_iota(jnp.uint32, (ROWS, 128), 0)
    l = jax.lax.broadcasted_iota(jnp.uint32, (ROWS, 128), 1)
    pm = (r * jnp.uint32(128) + l) % jnp.uint32(S)
    zero = jnp.uint32(0)
    a0 = xb
    a1 = jnp.where(pm >= jnp.uint32(1), shift(1), zero)
    a2 = jnp.where(pm >= jnp.uint32(2), shift(2), zero)
    a3 = jnp.where(pm >= jnp.uint32(3), shift(3), zero)
    ts = jnp.uint32(TABLE_SIZE)
    h2 = a1 * jnp.uint32(M1) + a0
    h3 = a2 * jnp.uint32(M2) + h2
    h4 = a3 * jnp.uint32(M3) + h3
    out[0] = (h2 % ts).astype(jnp.int32)
    out[1] = (h3 % ts).astype(jnp.int32)
    out[2] = (h4 % ts).astype(jnp.int32)


def _compute_indices(xf):
    return pl.pallas_call(
        _hash_body,
        out_shape=jax.ShapeDtypeStruct((3, ROWS, 128), jnp.int32),
    )(xf)


SEG = 62496
TAIL = TABLE_SIZE - 15 * SEG
NSTG = 4
STG = TAIL // NSTG
NB = 4
GB = CHUNK // NB


def _gather_body(idx_h, tb2_h, tb3_h, tb4_h, out_h,
                 idx_v0, idx_v1, out_v0, out_v1, bnc_v0, bnc_v1, tb_s,
                 sem_h, sem_s, sem_i, sem_g, sem_o):
    c = lax.axis_index("c")
    s = lax.axis_index("s")
    wid = s * 2 + c
    base = wid * CHUNK
    idx_bufs = (idx_v0, idx_v1)
    out_bufs = (out_v0, out_v1)
    bncs = (bnc_v0, bnc_v1)
    off = jnp.minimum(s * SEG, TABLE_SIZE - TAIL)
    tabs = (tb2_h, tb3_h, tb4_h)
    idx_descs = {}
    last_out = {0: None, 1: None}

    def idx_start(t, b):
        o = t * NTOT + base + b * GB
        idx_descs[(t, b)] = pltpu.async_copy(
            idx_h.at[pl.ds(o, GB)], idx_bufs[b % 2], sem_i
        )

    idx_start(0, 0)
    for t in range(3):
        tb_h = tabs[t]
        if t > 0:
            plsc.subcore_barrier()

        def stg_h(r):
            return pltpu.async_copy(
                tb_h.at[pl.ds(off + r * STG, STG)], bncs[r % 2], sem_h
            )

        hd = [None] * NSTG
        sd = [None] * NSTG
        hd[0] = stg_h(0)
        hd[1] = stg_h(1)
        for r in range(NSTG):
            hd[r].wait()
            sd[r] = pltpu.async_copy(
                bncs[r % 2], tb_s.at[pl.ds(off + r * STG, STG)], sem_s
            )
            if r + 2 < NSTG:
                sd[r].wait()
                hd[r + 2] = stg_h(r + 2)
        sd[NSTG - 2].wait()
        sd[NSTG - 1].wait()
        plsc.subcore_barrier()

        def out_start(b):
            o = t * NTOT + base + b * GB
            last_out[b % 2] = pltpu.async_copy(
                out_bufs[b % 2], out_h.at[pl.ds(o, GB)], sem_o
            )

        gd = {}
        for b in range(NB):
            idx_descs[(t, b)].wait()
            p = b % 2
            if last_out[p] is not None:
                last_out[p].wait()
            gd[b] = pltpu.async_copy(tb_s.at[idx_bufs[p]], out_bufs[p], sem_g)
            if b > 0:
                gd[b - 1].wait()
                out_start(b - 1)
            if b + 1 < NB:
                idx_start(t, b + 1)
            elif t < 2:
                idx_start(t + 1, 0)
        gd[NB - 1].wait()
        out_start(NB - 1)
    last_out[0].wait()
    last_out[1].wait()


@functools.cache
def _gather():
    return functools.partial(
        pl.kernel,
        out_type=jax.ShapeDtypeStruct((3 * NTOT,), jnp.float32),
        mesh=plsc.VectorSubcoreMesh(core_axis_name="c", subcore_axis_name="s"),
        scratch_types=[
            pltpu.VMEM((GB,), jnp.int32),
            pltpu.VMEM((GB,), jnp.int32),
            pltpu.VMEM((GB,), jnp.float32),
            pltpu.VMEM((GB,), jnp.float32),
            pltpu.VMEM((STG,), jnp.float32),
            pltpu.VMEM((STG,), jnp.float32),
            pltpu.VMEM_SHARED((TABLE_SIZE,), jnp.float32),
            pltpu.SemaphoreType.DMA,
            pltpu.SemaphoreType.DMA,
            pltpu.SemaphoreType.DMA,
            pltpu.SemaphoreType.DMA,
            pltpu.SemaphoreType.DMA,
        ],
    )(_gather_body)


def kernel(x, table_2, table_3, table_4):
    xf = x.reshape(ROWS, 128)
    idx = _compute_indices(xf).reshape(3 * NTOT)
    out = _gather()(idx, table_2, table_3, table_4)
    return out.reshape(3, B, S)


# speedup vs baseline: 1.0047x; 1.0004x over previous
import functools

import jax
import jax.numpy as jnp
from jax import lax
from jax.experimental import pallas as pl
from jax.experimental.pallas import tpu as pltpu
from jax.experimental.pallas import tpu_sc as plsc

B, S = 4096, 200
TABLE_SIZE = 1000000
MULT = 2654435761
M1 = MULT & 0xFFFFFFFF
M2 = (MULT * MULT) & 0xFFFFFFFF
M3 = (MULT * MULT * MULT) & 0xFFFFFFFF

NTOT = B * S
NW = 32
CHUNK = NTOT // NW
ROWS = NTOT // 128
HASH_BLK = 800


def _hash_body(x_ref, out):
    xb = x_ref[...].astype(jnp.uint32)
    zrow = jnp.zeros((1, 128), jnp.uint32)
    xprev = jnp.concatenate([zrow, xb[:-1, :]], axis=0)

    def shift(k):
        return jnp.concatenate([xprev[:, 128 - k:], xb[:, :128 - k]], axis=1)

    r = jax.lax.broadcasted_iota(jnp.uint32, (ROWS, 128), 0)
    l = jax.lax.broadcasted_iota(jnp.uint32, (ROWS, 128), 1)
    pm = (r * jnp.uint32(128) + l) % jnp.uint32(S)
    zero = jnp.uint32(0)
    a0 = xb
    a1 = jnp.where(pm >= jnp.uint32(1), shift(1), zero)
    a2 = jnp.where(pm >= jnp.uint32(2), shift(2), zero)
    a3 = jnp.where(pm >= jnp.uint32(3), shift(3), zero)
    ts = jnp.uint32(TABLE_SIZE)
    h2 = a1 * jnp.uint32(M1) + a0
    h3 = a2 * jnp.uint32(M2) + h2
    h4 = a3 * jnp.uint32(M3) + h3
    out[0] = (h2 % ts).astype(jnp.int32)
    out[1] = (h3 % ts).astype(jnp.int32)
    out[2] = (h4 % ts).astype(jnp.int32)


def _compute_indices(xf):
    return pl.pallas_call(
        _hash_body,
        out_shape=jax.ShapeDtypeStruct((3, ROWS, 128), jnp.int32),
        compiler_params=pltpu.CompilerParams(allow_input_fusion=[True]),
    )(xf)


SEG = 62496
TAIL = TABLE_SIZE - 15 * SEG
NSTG = 4
STG = TAIL // NSTG
NB = 4
GB = CHUNK // NB


def _gather_body(idx_h, tb2_h, tb3_h, tb4_h, out_h,
                 idx_v0, idx_v1, out_v0, out_v1, bnc_v0, bnc_v1, tb_s,
                 sem_h, sem_s, sem_i, sem_g, sem_o):
    c = lax.axis_index("c")
    s = lax.axis_index("s")
    wid = s * 2 + c
    base = wid * CHUNK
    idx_bufs = (idx_v0, idx_v1)
    out_bufs = (out_v0, out_v1)
    bncs = (bnc_v0, bnc_v1)
    off = jnp.minimum(s * SEG, TABLE_SIZE - TAIL)
    tabs = (tb2_h, tb3_h, tb4_h)
    idx_descs = {}
    last_out = {0: None, 1: None}

    def idx_start(t, b):
        o = t * NTOT + base + b * GB
        idx_descs[(t, b)] = pltpu.async_copy(
            idx_h.at[pl.ds(o, GB)], idx_bufs[b % 2], sem_i
        )

    idx_start(0, 0)
    for t in range(3):
        tb_h = tabs[t]
        if t > 0:
            plsc.subcore_barrier()

        def stg_h(r):
            return pltpu.async_copy(
                tb_h.at[pl.ds(off + r * STG, STG)], bncs[r % 2], sem_h
            )

        hd = [None] * NSTG
        sd = [None] * NSTG
        hd[0] = stg_h(0)
        hd[1] = stg_h(1)
        for r in range(NSTG):
            hd[r].wait()
            sd[r] = pltpu.async_copy(
                bncs[r % 2], tb_s.at[pl.ds(off + r * STG, STG)], sem_s
            )
            if r + 2 < NSTG:
                sd[r].wait()
                hd[r + 2] = stg_h(r + 2)
        sd[NSTG - 2].wait()
        sd[NSTG - 1].wait()
        plsc.subcore_barrier()

        def out_start(b):
            o = t * NTOT + base + b * GB
            last_out[b % 2] = pltpu.async_copy(
                out_bufs[b % 2], out_h.at[pl.ds(o, GB)], sem_o
            )

        gd = {}
        for b in range(NB):
            idx_descs[(t, b)].wait()
            p = b % 2
            if last_out[p] is not None:
                last_out[p].wait()
            gd[b] = pltpu.async_copy(tb_s.at[idx_bufs[p]], out_bufs[p], sem_g)
            if b > 0:
                gd[b - 1].wait()
                out_start(b - 1)
            if b + 1 < NB:
                idx_start(t, b + 1)
            elif t < 2:
                idx_start(t + 1, 0)
        gd[NB - 1].wait()
        out_start(NB - 1)
    last_out[0].wait()
    last_out[1].wait()


@functools.cache
def _gather():
    return functools.partial(
        pl.kernel,
        out_type=jax.ShapeDtypeStruct((3 * NTOT,), jnp.float32),
        mesh=plsc.VectorSubcoreMesh(core_axis_name="c", subcore_axis_name="s"),
        scratch_types=[
            pltpu.VMEM((GB,), jnp.int32),
            pltpu.VMEM((GB,), jnp.int32),
            pltpu.VMEM((GB,), jnp.float32),
            pltpu.VMEM((GB,), jnp.float32),
            pltpu.VMEM((STG,), jnp.float32),
            pltpu.VMEM((STG,), jnp.float32),
            pltpu.VMEM_SHARED((TABLE_SIZE,), jnp.float32),
            pltpu.SemaphoreType.DMA,
            pltpu.SemaphoreType.DMA,
            pltpu.SemaphoreType.DMA,
            pltpu.SemaphoreType.DMA,
            pltpu.SemaphoreType.DMA,
        ],
    )(_gather_body)


def kernel(x, table_2, table_3, table_4):
    xf = x.reshape(ROWS, 128)
    idx = _compute_indices(xf).reshape(3 * NTOT)
    out = _gather()(idx, table_2, table_3, table_4)
    return out.reshape(3, B, S)
